# baseline (speedup 1.0000x reference)
import functools

import jax
import jax.numpy as jnp
from jax import lax
from jax.experimental import pallas as pl
from jax.experimental.pallas import tpu as pltpu
from jax.experimental.pallas import tpu_sc as plsc

NN = 10000
EE = 320000
DIN = 128
HID = 32
NCLS = 40

NC = 2
NS = 16
NW = NC * NS
CH = 128
KCH = 80
EPT = CH * KCH
EPAD = EPT * NW
NPAD = 10240
RPS = NPAD // NS

_mesh = plsc.VectorSubcoreMesh(core_axis_name="c", subcore_axis_name="s")
_sc_params = pltpu.CompilerParams(use_tc_tiling_on_sc=False)


@functools.partial(
    pl.kernel,
    out_type=jax.ShapeDtypeStruct((NC * NPAD,), jnp.float32),
    mesh=_mesh,
    compiler_params=_sc_params,
    scratch_types=[
        pltpu.VMEM((KCH, CH), jnp.int32),
        pltpu.VMEM((CH,), jnp.float32),
        pltpu.VMEM((RPS,), jnp.float32),
        pltpu.VMEM_SHARED((NPAD,), jnp.float32),
        pltpu.SemaphoreType.DMA,
    ],
)
def _sc_degree(srcT_hbm, out_hbm, src_v, ones_v, zbuf_v, deg_sh, sem):
    cid = lax.axis_index("c")
    sid = lax.axis_index("s")
    wid = sid * NC + cid
    ones16 = jnp.ones((16,), jnp.float32)
    zero16 = jnp.zeros((16,), jnp.float32)
    for i in range(CH // 16):
        ones_v[pl.ds(i * 16, 16)] = ones16
    for i in range(RPS // 16):
        zbuf_v[pl.ds(i * 16, 16)] = zero16
    pltpu.sync_copy(srcT_hbm.at[wid], src_v)
    pltpu.sync_copy(zbuf_v, deg_sh.at[pl.ds(sid * RPS, RPS)])
    plsc.subcore_barrier()

    @pl.loop(0, KCH, step=8)
    def _chunk(j):
        for p in range(8):
            pltpu.async_copy(ones_v, deg_sh.at[src_v.at[j + p]], sem, add=True)
        for p in range(8):
            pltpu.make_async_copy(ones_v, deg_sh.at[src_v.at[j]], sem).wait()

    plsc.subcore_barrier()
    pltpu.sync_copy(deg_sh.at[pl.ds(sid * RPS, RPS)],
                    out_hbm.at[pl.ds(cid * NPAD + sid * RPS, RPS)])


@functools.partial(
    pl.kernel,
    out_type=jax.ShapeDtypeStruct((NC, NPAD, HID), jnp.float32),
    mesh=_mesh,
    compiler_params=_sc_params,
    scratch_types=[
        pltpu.VMEM((KCH, CH), jnp.int32),
        pltpu.VMEM((KCH, CH), jnp.int32),
        pltpu.VMEM((8, CH, HID), jnp.float32),
        pltpu.VMEM_SHARED((NPAD, HID), jnp.float32),
        pltpu.VMEM_SHARED((NN, HID), jnp.float32),
        pltpu.SemaphoreType.DMA,
        pltpu.SemaphoreType.DMA,
        pltpu.SemaphoreType.DMA,
        pltpu.SemaphoreType.DMA,
        pltpu.SemaphoreType.DMA,
        pltpu.SemaphoreType.DMA,
        pltpu.SemaphoreType.DMA,
        pltpu.SemaphoreType.DMA,
        pltpu.SemaphoreType.DMA,
        pltpu.SemaphoreType.DMA,
        pltpu.SemaphoreType.DMA,
        pltpu.SemaphoreType.DMA,
        pltpu.SemaphoreType.DMA,
        pltpu.SemaphoreType.DMA,
        pltpu.SemaphoreType.DMA,
        pltpu.SemaphoreType.DMA,
    ],
)
def _sc_propagate(z_hbm, srcT_hbm, dstT_hbm, zeros_hbm, out_hbm,
                  src_v, dst_v, rows_v, acc_sh, z_sh,
                  g0, g1, g2, g3, g4, g5, g6, g7,
                  s0, s1, s2, s3, s4, s5, s6, s7):
    gs = (g0, g1, g2, g3, g4, g5, g6, g7)
    ss = (s0, s1, s2, s3, s4, s5, s6, s7)
    cid = lax.axis_index("c")
    sid = lax.axis_index("s")
    wid = sid * NC + cid
    pltpu.sync_copy(srcT_hbm.at[wid], src_v)
    pltpu.sync_copy(dstT_hbm.at[wid], dst_v)
    pltpu.sync_copy(zeros_hbm.at[pl.ds(sid * RPS, RPS)],
                    acc_sh.at[pl.ds(sid * RPS, RPS)])
    zbase = jnp.minimum(sid * RPS, NN - RPS)
    pltpu.sync_copy(z_hbm.at[pl.ds(zbase, RPS)],
                    z_sh.at[pl.ds(zbase, RPS)])
    plsc.subcore_barrier()

    for c0 in range(4):
        pltpu.async_copy(z_sh.at[src_v.at[c0]], rows_v.at[c0], gs[c0])

    @pl.loop(0, KCH, step=8)
    def _oct(j):
        for p in range(8):
            c = j + p
            b = p
            b2 = (p + 4) % 8
            pltpu.make_async_copy(z_sh.at[src_v.at[c]], rows_v.at[b],
                                  gs[b]).wait()
            pltpu.async_copy(rows_v.at[b], acc_sh.at[dst_v.at[c]], ss[b],
                             add=True)

            @pl.when(c >= 4)
            def _drain():
                pltpu.make_async_copy(rows_v.at[b2],
                                      acc_sh.at[dst_v.at[c - 4]],
                                      ss[b2]).wait()

            @pl.when(c + 4 < KCH)
            def _prefetch():
                pltpu.async_copy(z_sh.at[src_v.at[c + 4]], rows_v.at[b2],
                                 gs[b2])

    for c in range(KCH - 4, KCH):
        b = c % 8
        pltpu.make_async_copy(rows_v.at[b], acc_sh.at[dst_v.at[c]],
                              ss[b]).wait()
    plsc.subcore_barrier()
    pltpu.sync_copy(acc_sh.at[pl.ds(sid * RPS, RPS)],
                    out_hbm.at[cid, pl.ds(sid * RPS, RPS)])


def _tc1_body(x_ref, w0_ref, w1_ref, b1_ref, deg_ref,
              xw0b_ref, z1_ref, dis_ref):
    deg = (deg_ref[pl.ds(0, NN)] + deg_ref[pl.ds(NPAD, NN)])
    dis = jnp.where(deg > 0, lax.rsqrt(jnp.maximum(deg, 1e-12)),
                    0.0).reshape(NN, 1)
    dis_ref[...] = dis
    x = x_ref[...]
    xw0b_ref[...] = (
        jnp.dot(x, w0_ref[...], preferred_element_type=jnp.float32) + b1_ref[...]
    )
    z1_ref[...] = dis * jnp.dot(x, w1_ref[...],
                                preferred_element_type=jnp.float32)


def _tc2_body(xw0b_ref, acc_ref, dis_ref, w02_ref, b2_ref,
              h_ref, z2_ref, hw0b_ref):
    dis = dis_ref[...]
    acc = acc_ref[0, :NN] + acc_ref[1, :NN]
    h = jnp.maximum(xw0b_ref[...] - dis * acc, 0.0)
    h_ref[...] = h
    z2_ref[...] = dis * h
    hw0b_ref[...] = (
        jnp.dot(h, w02_ref[...], preferred_element_type=jnp.float32)
        + b2_ref[...]
    )


def _tc3_body(hw0b_ref, acc_ref, dis_ref, w12_ref, logp_ref):
    dis = dis_ref[...]
    acc = acc_ref[0, :NN] + acc_ref[1, :NN]
    logits = hw0b_ref[...] - jnp.dot(
        dis * acc, w12_ref[...], preferred_element_type=jnp.float32
    )
    m = jnp.max(logits, axis=1, keepdims=True)
    lse = m + jnp.log(jnp.sum(jnp.exp(logits - m), axis=1, keepdims=True))
    logp_ref[...] = logits - lse


def kernel(x, edge_index, W0_1, W1_1, b1, W0_2, W1_2, b2):
    src = edge_index[0].astype(jnp.int32)
    dst = edge_index[1].astype(jnp.int32)
    pad = jnp.full((EPAD - EE,), NN, jnp.int32)
    srcT = jnp.concatenate([src, pad]).reshape(NW, KCH, CH)
    dstT = jnp.concatenate([dst, pad]).reshape(NW, KCH, CH)
    srcG = jnp.minimum(srcT, NN - 1)
    zeros2d = jnp.zeros((NPAD, HID), jnp.float32)

    deg_p = _sc_degree(srcT)

    xw0b, z1, dis_col = pl.pallas_call(
        _tc1_body,
        out_shape=(
            jax.ShapeDtypeStruct((NN, HID), jnp.float32),
            jax.ShapeDtypeStruct((NN, HID), jnp.float32),
            jax.ShapeDtypeStruct((NN, 1), jnp.float32),
        ),
    )(x, W0_1, W1_1, b1.reshape(1, HID), deg_p)

    acc1 = _sc_propagate(z1, srcG, dstT, zeros2d)

    h, z2, hw0b = pl.pallas_call(
        _tc2_body,
        out_shape=(
            jax.ShapeDtypeStruct((NN, HID), jnp.float32),
            jax.ShapeDtypeStruct((NN, HID), jnp.float32),
            jax.ShapeDtypeStruct((NN, NCLS), jnp.float32),
        ),
    )(xw0b, acc1, dis_col, W0_2, b2.reshape(1, NCLS))

    acc2 = _sc_propagate(z2, srcG, dstT, zeros2d)

    logp = pl.pallas_call(
        _tc3_body,
        out_shape=jax.ShapeDtypeStruct((NN, NCLS), jnp.float32),
    )(hw0b, acc2, dis_col, W1_2)

    return (logp, h)


# speedup vs baseline: 1.0311x; 1.0311x over previous
import functools

import jax
import jax.numpy as jnp
from jax import lax
from jax.experimental import pallas as pl
from jax.experimental.pallas import tpu as pltpu
from jax.experimental.pallas import tpu_sc as plsc

NN = 10000
EE = 320000
DIN = 128
HID = 32
NCLS = 40

NC = 2
NS = 16
NW = NC * NS
CH = 128
KCH = 80
EPT = CH * KCH
EPAD = EPT * NW
NPAD = 10240
RPS = NPAD // NS

_mesh = plsc.VectorSubcoreMesh(core_axis_name="c", subcore_axis_name="s")
_sc_params = pltpu.CompilerParams(use_tc_tiling_on_sc=False)


@functools.partial(
    pl.kernel,
    out_type=jax.ShapeDtypeStruct((NC * NPAD,), jnp.float32),
    mesh=_mesh,
    compiler_params=_sc_params,
    scratch_types=[
        pltpu.VMEM((KCH, CH), jnp.int32),
        pltpu.VMEM((CH,), jnp.float32),
        pltpu.VMEM((RPS,), jnp.float32),
        pltpu.VMEM_SHARED((NPAD,), jnp.float32),
        pltpu.SemaphoreType.DMA,
    ],
)
def _sc_degree(srcT_hbm, out_hbm, src_v, ones_v, zbuf_v, deg_sh, sem):
    cid = lax.axis_index("c")
    sid = lax.axis_index("s")
    wid = sid * NC + cid
    ones16 = jnp.ones((16,), jnp.float32)
    zero16 = jnp.zeros((16,), jnp.float32)
    for i in range(CH // 16):
        ones_v[pl.ds(i * 16, 16)] = ones16
    for i in range(RPS // 16):
        zbuf_v[pl.ds(i * 16, 16)] = zero16
    pltpu.sync_copy(srcT_hbm.at[wid], src_v)
    pltpu.sync_copy(zbuf_v, deg_sh.at[pl.ds(sid * RPS, RPS)])
    plsc.subcore_barrier()

    @pl.loop(0, KCH, step=8)
    def _chunk(j):
        for p in range(8):
            pltpu.async_copy(ones_v, deg_sh.at[src_v.at[j + p]], sem, add=True)
        for p in range(8):
            pltpu.make_async_copy(ones_v, deg_sh.at[src_v.at[j]], sem).wait()

    plsc.subcore_barrier()
    pltpu.sync_copy(deg_sh.at[pl.ds(sid * RPS, RPS)],
                    out_hbm.at[pl.ds(cid * NPAD + sid * RPS, RPS)])


@functools.partial(
    pl.kernel,
    out_type=jax.ShapeDtypeStruct((NC, NPAD, HID), jnp.float32),
    mesh=_mesh,
    compiler_params=_sc_params,
    scratch_types=[
        pltpu.VMEM((KCH, CH), jnp.int32),
        pltpu.VMEM((KCH, CH), jnp.int32),
        pltpu.VMEM((8, CH, HID), jnp.float32),
        pltpu.VMEM_SHARED((NPAD, HID), jnp.float32),
        pltpu.VMEM_SHARED((NPAD, HID), jnp.float32),
        pltpu.SemaphoreType.DMA,
        pltpu.SemaphoreType.DMA,
        pltpu.SemaphoreType.DMA,
        pltpu.SemaphoreType.DMA,
        pltpu.SemaphoreType.DMA,
        pltpu.SemaphoreType.DMA,
        pltpu.SemaphoreType.DMA,
        pltpu.SemaphoreType.DMA,
        pltpu.SemaphoreType.DMA,
        pltpu.SemaphoreType.DMA,
        pltpu.SemaphoreType.DMA,
        pltpu.SemaphoreType.DMA,
        pltpu.SemaphoreType.DMA,
        pltpu.SemaphoreType.DMA,
        pltpu.SemaphoreType.DMA,
        pltpu.SemaphoreType.DMA,
    ],
)
def _sc_propagate(z_hbm, srcT_hbm, dstT_hbm, zeros_hbm, out_hbm,
                  src_v, dst_v, rows_v, acc_sh, z_sh,
                  g0, g1, g2, g3, g4, g5, g6, g7,
                  s0, s1, s2, s3, s4, s5, s6, s7):
    gs = (g0, g1, g2, g3, g4, g5, g6, g7)
    ss = (s0, s1, s2, s3, s4, s5, s6, s7)
    cid = lax.axis_index("c")
    sid = lax.axis_index("s")
    wid = sid * NC + cid
    pltpu.sync_copy(srcT_hbm.at[wid], src_v)
    pltpu.sync_copy(dstT_hbm.at[wid], dst_v)
    pltpu.sync_copy(zeros_hbm.at[pl.ds(sid * RPS, RPS)],
                    acc_sh.at[pl.ds(sid * RPS, RPS)])
    pltpu.sync_copy(z_hbm.at[pl.ds(sid * RPS, RPS)],
                    z_sh.at[pl.ds(sid * RPS, RPS)])
    plsc.subcore_barrier()

    for c0 in range(4):
        pltpu.async_copy(z_sh.at[src_v.at[c0]], rows_v.at[c0], gs[c0])

    @pl.loop(0, KCH, step=8)
    def _oct(j):
        for p in range(8):
            c = j + p
            b = p
            b2 = (p + 4) % 8
            pltpu.make_async_copy(z_sh.at[src_v.at[c]], rows_v.at[b],
                                  gs[b]).wait()
            pltpu.async_copy(rows_v.at[b], acc_sh.at[dst_v.at[c]], ss[b],
                             add=True)

            @pl.when(c >= 4)
            def _drain():
                pltpu.make_async_copy(rows_v.at[b2],
                                      acc_sh.at[dst_v.at[c - 4]],
                                      ss[b2]).wait()

            @pl.when(c + 4 < KCH)
            def _prefetch():
                pltpu.async_copy(z_sh.at[src_v.at[c + 4]], rows_v.at[b2],
                                 gs[b2])

    for c in range(KCH - 4, KCH):
        b = c % 8
        pltpu.make_async_copy(rows_v.at[b], acc_sh.at[dst_v.at[c]],
                              ss[b]).wait()
    plsc.subcore_barrier()
    pltpu.sync_copy(acc_sh.at[pl.ds(sid * RPS, RPS)],
                    out_hbm.at[cid, pl.ds(sid * RPS, RPS)])


def _tc1_body(x_ref, w0_ref, w1_ref, b1_ref, deg_ref,
              xw0b_ref, z1_ref, dis_ref):
    deg = deg_ref[pl.ds(0, NPAD)] + deg_ref[pl.ds(NPAD, NPAD)]
    dis = jnp.where(deg > 0, lax.rsqrt(jnp.maximum(deg, 1e-12)),
                    0.0).reshape(NPAD, 1)
    dis_ref[...] = dis
    x = x_ref[...]
    xw0b_ref[...] = (
        jnp.dot(x, w0_ref[...], preferred_element_type=jnp.float32) + b1_ref[...]
    )
    z1 = dis[:NN] * jnp.dot(x, w1_ref[...], preferred_element_type=jnp.float32)
    z1_ref[...] = jnp.pad(z1, ((0, NPAD - NN), (0, 0)))


def _tc2_body(xw0b_ref, acc_ref, dis_ref, w02_ref, b2_ref,
              h_ref, z2_ref, hw0b_ref):
    dis = dis_ref[...]
    acc = acc_ref[0] + acc_ref[1]
    h = jnp.maximum(xw0b_ref[...] - (dis * acc)[:NN], 0.0)
    h_ref[...] = h
    z2_ref[...] = jnp.pad(dis[:NN] * h, ((0, NPAD - NN), (0, 0)))
    hw0b_ref[...] = (
        jnp.dot(h, w02_ref[...], preferred_element_type=jnp.float32)
        + b2_ref[...]
    )


def _tc3_body(hw0b_ref, acc_ref, dis_ref, w12_ref, logp_ref):
    dis = dis_ref[...]
    acc = acc_ref[0] + acc_ref[1]
    logits = hw0b_ref[...] - jnp.dot(
        (dis * acc)[:NN], w12_ref[...], preferred_element_type=jnp.float32
    )
    m = jnp.max(logits, axis=1, keepdims=True)
    lse = m + jnp.log(jnp.sum(jnp.exp(logits - m), axis=1, keepdims=True))
    logp_ref[...] = logits - lse


def kernel(x, edge_index, W0_1, W1_1, b1, W0_2, W1_2, b2):
    src = edge_index[0].astype(jnp.int32)
    dst = edge_index[1].astype(jnp.int32)
    pad = jnp.full((EPAD - EE,), NN, jnp.int32)
    srcT = jnp.concatenate([src, pad]).reshape(NW, KCH, CH)
    dstT = jnp.concatenate([dst, pad]).reshape(NW, KCH, CH)
    zeros2d = jnp.zeros((NPAD, HID), jnp.float32)

    deg_p = _sc_degree(srcT)

    xw0b, z1, dis_col = pl.pallas_call(
        _tc1_body,
        out_shape=(
            jax.ShapeDtypeStruct((NN, HID), jnp.float32),
            jax.ShapeDtypeStruct((NPAD, HID), jnp.float32),
            jax.ShapeDtypeStruct((NPAD, 1), jnp.float32),
        ),
    )(x, W0_1, W1_1, b1.reshape(1, HID), deg_p)

    acc1 = _sc_propagate(z1, srcT, dstT, zeros2d)

    h, z2, hw0b = pl.pallas_call(
        _tc2_body,
        out_shape=(
            jax.ShapeDtypeStruct((NN, HID), jnp.float32),
            jax.ShapeDtypeStruct((NPAD, HID), jnp.float32),
            jax.ShapeDtypeStruct((NN, NCLS), jnp.float32),
        ),
    )(xw0b, acc1, dis_col, W0_2, b2.reshape(1, NCLS))

    acc2 = _sc_propagate(z2, srcT, dstT, zeros2d)

    logp = pl.pallas_call(
        _tc3_body,
        out_shape=jax.ShapeDtypeStruct((NN, NCLS), jnp.float32),
    )(hw0b, acc2, dis_col, W1_2)

    return (logp, h)


# speedup vs baseline: 1.0402x; 1.0088x over previous
import functools

import jax
import jax.numpy as jnp
from jax import lax
from jax.experimental import pallas as pl
from jax.experimental.pallas import tpu as pltpu
from jax.experimental.pallas import tpu_sc as plsc

NN = 10000
EE = 320000
DIN = 128
HID = 32
NCLS = 40

NC = 2
NS = 16
NW = NC * NS
CH = 128
KCH = 80
EPT = CH * KCH
EPAD = EPT * NW
NPAD = 10240
RPS = NPAD // NS

_mesh = plsc.VectorSubcoreMesh(core_axis_name="c", subcore_axis_name="s")
_sc_params = pltpu.CompilerParams(use_tc_tiling_on_sc=False)


@functools.partial(
    pl.kernel,
    out_type=jax.ShapeDtypeStruct((NC * NPAD,), jnp.float32),
    mesh=_mesh,
    compiler_params=_sc_params,
    scratch_types=[
        pltpu.VMEM((KCH, CH), jnp.int32),
        pltpu.VMEM((CH,), jnp.float32),
        pltpu.VMEM((RPS,), jnp.float32),
        pltpu.VMEM_SHARED((NPAD,), jnp.float32),
        pltpu.SemaphoreType.DMA,
    ],
)
def _sc_degree(srcT_hbm, out_hbm, src_v, ones_v, zbuf_v, deg_sh, sem):
    cid = lax.axis_index("c")
    sid = lax.axis_index("s")
    wid = sid * NC + cid
    ones16 = jnp.ones((16,), jnp.float32)
    zero16 = jnp.zeros((16,), jnp.float32)
    for i in range(CH // 16):
        ones_v[pl.ds(i * 16, 16)] = ones16
    for i in range(RPS // 16):
        zbuf_v[pl.ds(i * 16, 16)] = zero16
    pltpu.sync_copy(srcT_hbm.at[wid], src_v)
    pltpu.sync_copy(zbuf_v, deg_sh.at[pl.ds(sid * RPS, RPS)])
    plsc.subcore_barrier()

    @pl.loop(0, KCH, step=8)
    def _chunk(j):
        for p in range(8):
            pltpu.async_copy(ones_v, deg_sh.at[src_v.at[j + p]], sem, add=True)
        for p in range(8):
            pltpu.make_async_copy(ones_v, deg_sh.at[src_v.at[j]], sem).wait()

    plsc.subcore_barrier()
    pltpu.sync_copy(deg_sh.at[pl.ds(sid * RPS, RPS)],
                    out_hbm.at[pl.ds(cid * NPAD + sid * RPS, RPS)])


@functools.partial(
    pl.kernel,
    out_type=jax.ShapeDtypeStruct((NC, NPAD, HID), jnp.float32),
    mesh=_mesh,
    compiler_params=_sc_params,
    scratch_types=[
        pltpu.VMEM((KCH, CH), jnp.int32),
        pltpu.VMEM((KCH, CH), jnp.int32),
        pltpu.VMEM((8, CH, HID), jnp.float32),
        pltpu.VMEM_SHARED((NPAD, HID), jnp.float32),
        pltpu.VMEM_SHARED((NPAD, HID), jnp.float32),
        pltpu.SemaphoreType.DMA,
        pltpu.SemaphoreType.DMA,
        pltpu.SemaphoreType.DMA,
        pltpu.SemaphoreType.DMA,
        pltpu.SemaphoreType.DMA,
        pltpu.SemaphoreType.DMA,
        pltpu.SemaphoreType.DMA,
        pltpu.SemaphoreType.DMA,
        pltpu.SemaphoreType.DMA,
        pltpu.SemaphoreType.DMA,
        pltpu.SemaphoreType.DMA,
        pltpu.SemaphoreType.DMA,
        pltpu.SemaphoreType.DMA,
        pltpu.SemaphoreType.DMA,
        pltpu.SemaphoreType.DMA,
        pltpu.SemaphoreType.DMA,
    ],
)
def _sc_propagate(z_hbm, srcT_hbm, dstT_hbm, zeros_hbm, out_hbm,
                  src_v, dst_v, rows_v, acc_sh, z_sh,
                  g0, g1, g2, g3, g4, g5, g6, g7,
                  s0, s1, s2, s3, s4, s5, s6, s7):
    gs = (g0, g1, g2, g3, g4, g5, g6, g7)
    ss = (s0, s1, s2, s3, s4, s5, s6, s7)
    cid = lax.axis_index("c")
    sid = lax.axis_index("s")
    wid = sid * NC + cid
    pltpu.sync_copy(srcT_hbm.at[wid], src_v)
    pltpu.sync_copy(dstT_hbm.at[wid], dst_v)
    pltpu.sync_copy(zeros_hbm.at[pl.ds(sid * RPS, RPS)],
                    acc_sh.at[pl.ds(sid * RPS, RPS)])
    pltpu.sync_copy(z_hbm.at[pl.ds(sid * RPS, RPS)],
                    z_sh.at[pl.ds(sid * RPS, RPS)])
    plsc.subcore_barrier()

    for c0 in range(4):
        pltpu.async_copy(z_sh.at[src_v.at[c0]], rows_v.at[c0], gs[c0])

    @pl.loop(0, KCH, step=8)
    def _oct(j):
        for p in range(8):
            c = j + p
            b = p
            b2 = (p + 4) % 8
            pltpu.make_async_copy(z_sh.at[src_v.at[c]], rows_v.at[b],
                                  gs[b]).wait()
            pltpu.async_copy(rows_v.at[b], acc_sh.at[dst_v.at[c]], ss[b],
                             add=True)

            @pl.when(c >= 4)
            def _drain():
                pltpu.make_async_copy(rows_v.at[b2],
                                      acc_sh.at[dst_v.at[c - 4]],
                                      ss[b2]).wait()

            @pl.when(c + 4 < KCH)
            def _prefetch():
                pltpu.async_copy(z_sh.at[src_v.at[c + 4]], rows_v.at[b2],
                                 gs[b2])

    for c in range(KCH - 4, KCH):
        b = c % 8
        pltpu.make_async_copy(rows_v.at[b], acc_sh.at[dst_v.at[c]],
                              ss[b]).wait()
    plsc.subcore_barrier()
    pltpu.sync_copy(acc_sh.at[pl.ds(sid * RPS, RPS)],
                    out_hbm.at[cid, pl.ds(sid * RPS, RPS)])


def _tc1a_body(x_ref, w0_ref, w1_ref, b1_ref, xw0b_ref, u1_ref):
    x = x_ref[...]
    xw0b_ref[...] = (
        jnp.dot(x, w0_ref[...], preferred_element_type=jnp.float32) + b1_ref[...]
    )
    u1_ref[...] = jnp.dot(x, w1_ref[...], preferred_element_type=jnp.float32)


def _tc1b_body(u1_ref, deg_ref, z1_ref, dis_ref):
    deg = deg_ref[pl.ds(0, NPAD)] + deg_ref[pl.ds(NPAD, NPAD)]
    dis = jnp.where(deg > 0, lax.rsqrt(jnp.maximum(deg, 1e-12)),
                    0.0).reshape(NPAD, 1)
    dis_ref[...] = dis
    z1 = dis[:NN] * u1_ref[...]
    z1_ref[...] = jnp.pad(z1, ((0, NPAD - NN), (0, 0)))


def _tc2_body(xw0b_ref, acc_ref, dis_ref, w02_ref, b2_ref,
              h_ref, z2_ref, hw0b_ref):
    dis = dis_ref[...]
    acc = acc_ref[0] + acc_ref[1]
    h = jnp.maximum(xw0b_ref[...] - (dis * acc)[:NN], 0.0)
    h_ref[...] = h
    z2_ref[...] = jnp.pad(dis[:NN] * h, ((0, NPAD - NN), (0, 0)))
    hw0b_ref[...] = (
        jnp.dot(h, w02_ref[...], preferred_element_type=jnp.float32)
        + b2_ref[...]
    )


def _tc3_body(hw0b_ref, acc_ref, dis_ref, w12_ref, logp_ref):
    dis = dis_ref[...]
    acc = acc_ref[0] + acc_ref[1]
    logits = hw0b_ref[...] - jnp.dot(
        (dis * acc)[:NN], w12_ref[...], preferred_element_type=jnp.float32
    )
    m = jnp.max(logits, axis=1, keepdims=True)
    lse = m + jnp.log(jnp.sum(jnp.exp(logits - m), axis=1, keepdims=True))
    logp_ref[...] = logits - lse


def kernel(x, edge_index, W0_1, W1_1, b1, W0_2, W1_2, b2):
    src = edge_index[0].astype(jnp.int32)
    dst = edge_index[1].astype(jnp.int32)
    pad = jnp.full((EPAD - EE,), NN, jnp.int32)
    srcT = jnp.concatenate([src, pad]).reshape(NW, KCH, CH)
    dstT = jnp.concatenate([dst, pad]).reshape(NW, KCH, CH)
    zeros2d = jnp.zeros((NPAD, HID), jnp.float32)

    deg_p = _sc_degree(srcT)

    xw0b, u1 = pl.pallas_call(
        _tc1a_body,
        out_shape=(
            jax.ShapeDtypeStruct((NN, HID), jnp.float32),
            jax.ShapeDtypeStruct((NN, HID), jnp.float32),
        ),
    )(x, W0_1, W1_1, b1.reshape(1, HID))

    z1, dis_col = pl.pallas_call(
        _tc1b_body,
        out_shape=(
            jax.ShapeDtypeStruct((NPAD, HID), jnp.float32),
            jax.ShapeDtypeStruct((NPAD, 1), jnp.float32),
        ),
    )(u1, deg_p)

    acc1 = _sc_propagate(z1, srcT, dstT, zeros2d)

    h, z2, hw0b = pl.pallas_call(
        _tc2_body,
        out_shape=(
            jax.ShapeDtypeStruct((NN, HID), jnp.float32),
            jax.ShapeDtypeStruct((NPAD, HID), jnp.float32),
            jax.ShapeDtypeStruct((NN, NCLS), jnp.float32),
        ),
    )(xw0b, acc1, dis_col, W0_2, b2.reshape(1, NCLS))

    acc2 = _sc_propagate(z2, srcT, dstT, zeros2d)

    logp = pl.pallas_call(
        _tc3_body,
        out_shape=jax.ShapeDtypeStruct((NN, NCLS), jnp.float32),
    )(hw0b, acc2, dis_col, W1_2)

    return (logp, h)
